# baseline (speedup 1.0000x reference)
import functools
import math

import jax
import jax.numpy as jnp
from jax import lax
from jax.experimental import pallas as pl
from jax.experimental.pallas import tpu as pltpu
from jax.experimental.pallas import tpu_sc as plsc

_NC, _NS, _L = 2, 16, 16
_NW = _NC * _NS


def _sc_scores(bigram_flat, samples, start, end):
    S, N = samples.shape
    rpw = S // _NW
    n_batches = N // 128
    mesh = plsc.VectorSubcoreMesh(core_axis_name="c", subcore_axis_name="s")

    def body(bigram_hbm, samples_hbm, start_hbm, end_hbm,
             w_hbm, gold_hbm,
             row0_v, row1_v, row2_v, idx0_v, idx1_v, idx2_v,
             val0_v, val1_v, val2_v,
             sv_v, ev_v, w_v, g_v, gidx_v, gval_v,
             gsem0, gsem1, gsem2, rsem, gold_sem):
        cid = lax.axis_index("c")
        sid = lax.axis_index("s")
        wid = sid * _NC + cid
        lanes = jax.lax.iota(jnp.int32, _L)
        rows = [row0_v, row1_v, row2_v]
        idxs = [idx0_v, idx1_v, idx2_v]
        vals = [val0_v, val1_v, val2_v]
        gsems = [gsem0, gsem1, gsem2]

        pltpu.sync_copy(start_hbm, sv_v.at[pl.ds(0, N)])
        pltpu.sync_copy(end_hbm, ev_v.at[pl.ds(0, N)])
        row0_v[pl.ds(N, _L)] = jnp.zeros((_L,), jnp.int32)
        row1_v[pl.ds(N, _L)] = jnp.zeros((_L,), jnp.int32)
        row2_v[pl.ds(N, _L)] = jnp.zeros((_L,), jnp.int32)

        def compute_indices(row_v, idx_v):
            def batch(j, _):
                for c in range(128 // _L):
                    base = j * 128 + c * _L
                    a = row_v[pl.ds(base, _L)]
                    b = row_v[pl.ds(base + 1, _L)]
                    idx_v[pl.ds(base, _L)] = (
                        ((a >> 3) << 15) + ((b >> 7) << 10)
                        + ((a & 7) << 7) + (b & 127)
                    )
                return 0
            lax.fori_loop(0, n_batches, batch, 0)
            t0 = row_v[pl.ds(0, _L)][0]
            tN = row_v[pl.ds(N - _L, _L)][_L - 1]
            return sv_v[pl.ds(t0, _L)][0] + ev_v[pl.ds(tN, _L)][0]

        def sum_vals(val_v, se):
            def batch(j, acc):
                for c in range(128 // _L):
                    acc = acc + val_v[pl.ds(j * 128 + c * _L, _L)]
                return acc
            acc = lax.fori_loop(0, n_batches, batch,
                                jnp.zeros((_L,), jnp.float32))
            corr = val_v[pl.ds(N - _L, _L)][_L - 1]
            return jnp.sum(acc) - corr + se

        gbase = wid * 128
        for c in range(128 // _L):
            i = lanes + (gbase + c * _L)
            ip1 = i + 1
            tiled = (((i >> 3) << 15) + ((ip1 >> 7) << 10)
                     + ((i & 7) << 7) + (ip1 & 127))
            gidx_v[pl.ds(c * _L, _L)] = jnp.where(i < N - 1, tiled, 0)
        gold_d = pltpu.async_copy(bigram_hbm.at[gidx_v], gval_v, gold_sem)

        D = 3
        base_row = wid * rpw
        pltpu.sync_copy(samples_hbm.at[base_row], rows[0].at[pl.ds(0, N)])
        se = [None] * D
        gd = [None] * D
        rd = None
        if rpw > 1:
            rd = pltpu.async_copy(samples_hbm.at[base_row + 1],
                                  rows[1].at[pl.ds(0, N)], rsem)
        se[0] = compute_indices(rows[0], idxs[0])
        gd[0] = pltpu.async_copy(bigram_hbm.at[idxs[0]], vals[0], gsems[0])
        for k in range(1, D):
            if k < rpw:
                rd.wait()
                if k + 1 < rpw:
                    rd = pltpu.async_copy(samples_hbm.at[base_row + k + 1],
                                          rows[(k + 1) % D].at[pl.ds(0, N)],
                                          rsem)
                se[k] = compute_indices(rows[k], idxs[k])
                gd[k] = pltpu.async_copy(bigram_hbm.at[idxs[k]], vals[k],
                                         gsems[k])
        for r in range(rpw):
            s = r % D
            se_cur = se[s]
            gd[s].wait()
            w = sum_vals(vals[s], se_cur)
            plsc.store_scatter(w_v, [jnp.full((_L,), r, jnp.int32)],
                               jnp.full((_L,), w, jnp.float32),
                               mask=lanes == 0)
            k = r + D
            if k < rpw:
                rd.wait()
                if k + 1 < rpw:
                    rd = pltpu.async_copy(samples_hbm.at[base_row + k + 1],
                                          rows[(k + 1) % D].at[pl.ds(0, N)],
                                          rsem)
                se[s] = compute_indices(rows[s], idxs[s])
                gd[s] = pltpu.async_copy(bigram_hbm.at[idxs[s]], vals[s],
                                         gsems[s])
        pltpu.sync_copy(w_v, w_hbm.at[pl.ds(base_row, rpw)])

        gold_d.wait()
        gacc = jnp.zeros((_L,), jnp.float32)
        for c in range(128 // _L):
            i = lanes + (gbase + c * _L)
            gacc = gacc + jnp.where(i < N - 1, gval_v[pl.ds(c * _L, _L)], 0.0)
        gpart = jnp.sum(gacc)
        gpart = jnp.where(
            wid == 0,
            gpart + sv_v[pl.ds(0, _L)][0] + ev_v[pl.ds(N - _L, _L)][_L - 1],
            gpart)
        g_v[...] = jnp.where(lanes == 0, gpart, 0.0)
        pltpu.sync_copy(g_v, gold_hbm.at[wid])

    f = pl.kernel(
        body,
        out_type=(
            jax.ShapeDtypeStruct((S,), jnp.float32),
            jax.ShapeDtypeStruct((_NW, _L), jnp.float32),
        ),
        mesh=mesh,
        scratch_types=[
            pltpu.VMEM((N + _L,), jnp.int32),
            pltpu.VMEM((N + _L,), jnp.int32),
            pltpu.VMEM((N + _L,), jnp.int32),
            pltpu.VMEM((N,), jnp.int32),
            pltpu.VMEM((N,), jnp.int32),
            pltpu.VMEM((N,), jnp.int32),
            pltpu.VMEM((N,), jnp.float32),
            pltpu.VMEM((N,), jnp.float32),
            pltpu.VMEM((N,), jnp.float32),
            pltpu.VMEM((N + _L,), jnp.float32),
            pltpu.VMEM((N + _L,), jnp.float32),
            pltpu.VMEM((rpw,), jnp.float32),
            pltpu.VMEM((_L,), jnp.float32),
            pltpu.VMEM((128,), jnp.int32),
            pltpu.VMEM((128,), jnp.float32),
            pltpu.SemaphoreType.DMA,
            pltpu.SemaphoreType.DMA,
            pltpu.SemaphoreType.DMA,
            pltpu.SemaphoreType.DMA,
            pltpu.SemaphoreType.DMA,
        ],
        compiler_params=pltpu.CompilerParams(needs_layout_passes=False),
    )
    return f(bigram_flat, samples, start, end)


def _tc_finish(w2d, gparts, n_words):
    log_fact = math.lgamma(n_words + 1)
    log_n = math.log(n_words)

    def body(w_ref, g_ref, loss_ref, cnt_ref):
        wv = w_ref[...]
        g = jnp.sum(g_ref[...])
        m = jnp.max(wv)
        lse = m + jnp.log(jnp.sum(jnp.exp(wv - m)))
        loss_ref[0, 0] = -g + jnp.float32(log_fact - log_n) + lse
        cnt_ref[0, 0] = jnp.sum((g > wv).astype(jnp.int32))

    return pl.pallas_call(
        body,
        out_shape=(
            jax.ShapeDtypeStruct((1, 1), jnp.float32),
            jax.ShapeDtypeStruct((1, 1), jnp.int32),
        ),
        in_specs=[
            pl.BlockSpec(memory_space=pltpu.VMEM),
            pl.BlockSpec(memory_space=pltpu.VMEM),
        ],
        out_specs=(
            pl.BlockSpec(memory_space=pltpu.SMEM),
            pl.BlockSpec(memory_space=pltpu.SMEM),
        ),
    )(w2d, gparts)


def kernel(bigram, start, end, samples):
    n_words = start.shape[0]
    S = samples.shape[0]
    bigram_t = (bigram.reshape(n_words // 8, 8, n_words // 128, 128)
                .transpose(0, 2, 1, 3).reshape(-1))
    w, gparts = _sc_scores(bigram_t, samples, start, end)
    return (w[0], w[1].astype(jnp.int32))


# speedup vs baseline: 1.0077x; 1.0077x over previous
import math

import jax
import jax.numpy as jnp
from jax import lax
from jax.experimental import pallas as pl
from jax.experimental.pallas import tpu as pltpu
from jax.experimental.pallas import tpu_sc as plsc

_NC, _NS, _L = 2, 16, 16
_NW = _NC * _NS


def _sc_scores(bigram_flat, samples, start, end):
    S, N = samples.shape
    rpw = S // _NW
    n_batches = N // 128
    mesh = plsc.VectorSubcoreMesh(core_axis_name="c", subcore_axis_name="s")

    def body(bigram_hbm, samples_hbm, start_hbm, end_hbm,
             w_hbm, gold_hbm,
             row0_v, row1_v, row2_v, idx0_v, idx1_v, idx2_v,
             val0_v, val1_v, val2_v,
             sv_v, ev_v, w_v, g_v, gidx_v, gval_v,
             gsem0, gsem1, gsem2, rsem, gold_sem):
        cid = lax.axis_index("c")
        sid = lax.axis_index("s")
        wid = sid * _NC + cid
        lanes = jax.lax.iota(jnp.int32, _L)
        rows = [row0_v, row1_v, row2_v]
        idxs = [idx0_v, idx1_v, idx2_v]
        vals = [val0_v, val1_v, val2_v]
        gsems = [gsem0, gsem1, gsem2]

        pltpu.sync_copy(start_hbm, sv_v.at[pl.ds(0, N)])
        pltpu.sync_copy(end_hbm, ev_v.at[pl.ds(0, N)])
        row0_v[pl.ds(N, _L)] = jnp.zeros((_L,), jnp.int32)
        row1_v[pl.ds(N, _L)] = jnp.zeros((_L,), jnp.int32)
        row2_v[pl.ds(N, _L)] = jnp.zeros((_L,), jnp.int32)

        def compute_indices(row_v, idx_v):
            def batch(j, _):
                for c in range(128 // _L):
                    base = j * 128 + c * _L
                    a = row_v[pl.ds(base, _L)]
                    b = row_v[pl.ds(base + 1, _L)]
                    idx_v[pl.ds(base, _L)] = (
                        ((a >> 3) << 15) + ((b >> 7) << 10)
                        + ((a & 7) << 7) + (b & 127)
                    )
                return 0
            lax.fori_loop(0, n_batches, batch, 0)
            t0 = row_v[pl.ds(0, _L)][0]
            tN = row_v[pl.ds(N - _L, _L)][_L - 1]
            return sv_v[pl.ds(t0, _L)][0] + ev_v[pl.ds(tN, _L)][0]

        def sum_vals(val_v, se):
            def batch(j, acc):
                for c in range(128 // _L):
                    acc = acc + val_v[pl.ds(j * 128 + c * _L, _L)]
                return acc
            acc = lax.fori_loop(0, n_batches, batch,
                                jnp.zeros((_L,), jnp.float32))
            corr = val_v[pl.ds(N - _L, _L)][_L - 1]
            return jnp.sum(acc) - corr + se

        gbase = wid * 128
        for c in range(128 // _L):
            i = lanes + (gbase + c * _L)
            ip1 = i + 1
            tiled = (((i >> 3) << 15) + ((ip1 >> 7) << 10)
                     + ((i & 7) << 7) + (ip1 & 127))
            gidx_v[pl.ds(c * _L, _L)] = jnp.where(i < N - 1, tiled, 0)
        gold_d = pltpu.async_copy(bigram_hbm.at[gidx_v], gval_v, gold_sem)

        D = 3
        base_row = wid * rpw
        pltpu.sync_copy(samples_hbm.at[base_row], rows[0].at[pl.ds(0, N)])
        se = [None] * D
        gd = [None] * D
        rd = None
        if rpw > 1:
            rd = pltpu.async_copy(samples_hbm.at[base_row + 1],
                                  rows[1].at[pl.ds(0, N)], rsem)
        se[0] = compute_indices(rows[0], idxs[0])
        gd[0] = pltpu.async_copy(bigram_hbm.at[idxs[0]], vals[0], gsems[0])
        for k in range(1, D):
            if k < rpw:
                rd.wait()
                if k + 1 < rpw:
                    rd = pltpu.async_copy(samples_hbm.at[base_row + k + 1],
                                          rows[(k + 1) % D].at[pl.ds(0, N)],
                                          rsem)
                se[k] = compute_indices(rows[k], idxs[k])
                gd[k] = pltpu.async_copy(bigram_hbm.at[idxs[k]], vals[k],
                                         gsems[k])
        for r in range(rpw):
            s = r % D
            se_cur = se[s]
            gd[s].wait()
            w = sum_vals(vals[s], se_cur)
            plsc.store_scatter(w_v, [jnp.full((_L,), r, jnp.int32)],
                               jnp.full((_L,), w, jnp.float32),
                               mask=lanes == 0)
            k = r + D
            if k < rpw:
                rd.wait()
                if k + 1 < rpw:
                    rd = pltpu.async_copy(samples_hbm.at[base_row + k + 1],
                                          rows[(k + 1) % D].at[pl.ds(0, N)],
                                          rsem)
                se[s] = compute_indices(rows[s], idxs[s])
                gd[s] = pltpu.async_copy(bigram_hbm.at[idxs[s]], vals[s],
                                         gsems[s])
        pltpu.sync_copy(w_v, w_hbm.at[pl.ds(base_row, rpw)])

        gold_d.wait()
        gacc = jnp.zeros((_L,), jnp.float32)
        for c in range(128 // _L):
            i = lanes + (gbase + c * _L)
            gacc = gacc + jnp.where(i < N - 1, gval_v[pl.ds(c * _L, _L)], 0.0)
        gpart = jnp.sum(gacc)
        gpart = jnp.where(
            wid == 0,
            gpart + sv_v[pl.ds(0, _L)][0] + ev_v[pl.ds(N - _L, _L)][_L - 1],
            gpart)
        g_v[...] = jnp.where(lanes == 0, gpart, 0.0)
        pltpu.sync_copy(g_v, gold_hbm.at[wid])

    f = pl.kernel(
        body,
        out_type=(
            jax.ShapeDtypeStruct((S,), jnp.float32),
            jax.ShapeDtypeStruct((_NW, _L), jnp.float32),
        ),
        mesh=mesh,
        scratch_types=[
            pltpu.VMEM((N + _L,), jnp.int32),
            pltpu.VMEM((N + _L,), jnp.int32),
            pltpu.VMEM((N + _L,), jnp.int32),
            pltpu.VMEM((N,), jnp.int32),
            pltpu.VMEM((N,), jnp.int32),
            pltpu.VMEM((N,), jnp.int32),
            pltpu.VMEM((N,), jnp.float32),
            pltpu.VMEM((N,), jnp.float32),
            pltpu.VMEM((N,), jnp.float32),
            pltpu.VMEM((N + _L,), jnp.float32),
            pltpu.VMEM((N + _L,), jnp.float32),
            pltpu.VMEM((rpw,), jnp.float32),
            pltpu.VMEM((_L,), jnp.float32),
            pltpu.VMEM((128,), jnp.int32),
            pltpu.VMEM((128,), jnp.float32),
            pltpu.SemaphoreType.DMA,
            pltpu.SemaphoreType.DMA,
            pltpu.SemaphoreType.DMA,
            pltpu.SemaphoreType.DMA,
            pltpu.SemaphoreType.DMA,
        ],
        compiler_params=pltpu.CompilerParams(needs_layout_passes=False),
    )
    return f(bigram_flat, samples, start, end)


def _tc_finish(w2d, gparts, n_words):
    log_fact = math.lgamma(n_words + 1)
    log_n = math.log(n_words)

    def body(w_ref, g_ref, loss_ref, cnt_ref):
        wv = w_ref[...]
        g = jnp.sum(g_ref[...])
        m = jnp.max(wv)
        lse = m + jnp.log(jnp.sum(jnp.exp(wv - m)))
        loss_ref[0, 0] = -g + jnp.float32(log_fact - log_n) + lse
        cnt_ref[0, 0] = jnp.sum((g > wv).astype(jnp.int32))

    return pl.pallas_call(
        body,
        out_shape=(
            jax.ShapeDtypeStruct((1, 1), jnp.float32),
            jax.ShapeDtypeStruct((1, 1), jnp.int32),
        ),
        in_specs=[
            pl.BlockSpec(memory_space=pltpu.VMEM),
            pl.BlockSpec(memory_space=pltpu.VMEM),
        ],
        out_specs=(
            pl.BlockSpec(memory_space=pltpu.SMEM),
            pl.BlockSpec(memory_space=pltpu.SMEM),
        ),
    )(w2d, gparts)


def kernel(bigram, start, end, samples):
    n_words = start.shape[0]
    S = samples.shape[0]
    bigram_t = (bigram.reshape(n_words // 8, 8, n_words // 128, 128)
                .transpose(0, 2, 1, 3).reshape(-1))
    w, gparts = _sc_scores(bigram_t, samples, start, end)
    loss, cnt = _tc_finish(w.reshape(S // 128, 128), gparts, n_words)
    return (loss.reshape(()), cnt.reshape(()))
